# Initial kernel scaffold; baseline (speedup 1.0000x reference)
import jax
import jax.numpy as jnp
from jax.experimental import pallas as pl


def kernel(x, router_w, w1, w2):
    raise NotImplementedError("write your pallas kernel here")


# speedup vs baseline: 1.2477x; 1.2477x over previous
import jax
import jax.numpy as jnp
from jax import lax
from jax.experimental import pallas as pl
from jax.experimental.pallas import tpu as pltpu
from jax.experimental.pallas import tpu_sc as plsc

E = 8
K = 2
D = 768
BS = 128
DFFN = 1536
T = 2048
NPAIR = T * K
NB = 40
P = NB * BS
TBLK = 128
NTB = T // TBLK
NW = 32


def _router_body(x_ref, rwt_ref, sel_ref, rank_ref, rw_ref, cnt0_ref,
                 offs_ref, be_ref, act_ref, carry0, carry1):
    i = pl.program_id(0)

    @pl.when(i == 0)
    def _():
        carry0[...] = jnp.zeros_like(carry0)
        carry1[...] = jnp.zeros_like(carry1)

    xb = x_ref[...]
    logits = jnp.dot(xb, rwt_ref[...], preferred_element_type=jnp.float32)
    m = jnp.max(logits, axis=1, keepdims=True)
    ex = jnp.exp(logits - m)
    p = ex / jnp.sum(ex, axis=1, keepdims=True)
    lane8 = lax.broadcasted_iota(jnp.int32, (TBLK, E), 1)
    m1 = jnp.max(p, axis=1, keepdims=True)
    i1 = jnp.min(jnp.where(p >= m1, lane8, E), axis=1, keepdims=True)
    p2 = jnp.where(lane8 == i1, -1.0, p)
    m2 = jnp.max(p2, axis=1, keepdims=True)
    i2 = jnp.min(jnp.where(p2 >= m2, lane8, E), axis=1, keepdims=True)
    ssum = m1 + m2
    sel_ref[0] = i1
    sel_ref[1] = i2
    rw_ref[0] = m1 / ssum
    rw_ref[1] = m2 / ssum

    lane128 = lax.broadcasted_iota(jnp.int32, (TBLK, 128), 1)
    row128 = lax.broadcasted_iota(jnp.int32, (TBLK, 128), 0)
    tril = (row128 >= lane128).astype(jnp.float32)
    for g, (sel, carry) in enumerate(((i1, carry0), (i2, carry1))):
        oh = (sel == lane128).astype(jnp.float32)
        cum = jnp.dot(tril, oh, preferred_element_type=jnp.float32)
        cb = carry[...]
        rank = jnp.sum(oh * (cum + cb - 1.0), axis=1, keepdims=True)
        rank_ref[g] = rank.astype(jnp.int32)
        carry[...] = cb + jnp.sum(oh, axis=0, keepdims=True)

    @pl.when(i == NTB - 1)
    def _():
        c0 = carry0[...]
        tot = (c0 + carry1[...]).astype(jnp.int32)
        padded = ((tot + 127) >> 7) << 7
        nblk = (padded >> 7).astype(jnp.float32)
        mstrict = (row128 < lane128).astype(jnp.float32)
        mincl = (row128 <= lane128).astype(jnp.float32)
        offs = jnp.dot(padded.astype(jnp.float32), mstrict,
                       preferred_element_type=jnp.float32)
        cnt0_ref[...] = c0.astype(jnp.int32).reshape(1, 1, 128)
        offs_ref[...] = offs.astype(jnp.int32).reshape(1, 1, 128)
        bo = jnp.dot(nblk, mincl, preferred_element_type=jnp.float32)
        lm = lane128 < E
        cmp = jnp.logical_and(row128.astype(jnp.float32) >= bo, lm)
        be = jnp.sum(cmp.astype(jnp.float32), axis=1, keepdims=True)
        be_ref[...] = jnp.minimum(be, float(E - 1)).astype(jnp.int32)
        nbtot = jnp.sum(jnp.where(lm[0:1, :], nblk, 0.0), axis=1,
                        keepdims=True)
        act_ref[...] = (row128[:, 0:1].astype(jnp.float32) < nbtot
                        ).astype(jnp.int32)


def _router(x2d, rwt):
    return pl.pallas_call(
        _router_body,
        grid=(NTB,),
        in_specs=[
            pl.BlockSpec((TBLK, D), lambda i: (i, 0)),
            pl.BlockSpec((D, E), lambda i: (0, 0)),
        ],
        out_specs=[
            pl.BlockSpec((K, TBLK, 1), lambda i: (0, i, 0)),
            pl.BlockSpec((K, TBLK, 1), lambda i: (0, i, 0)),
            pl.BlockSpec((K, TBLK, 1), lambda i: (0, i, 0)),
            pl.BlockSpec((1, 1, 128), lambda i: (0, 0, 0)),
            pl.BlockSpec((1, 1, 128), lambda i: (0, 0, 0)),
            pl.BlockSpec((128, 1), lambda i: (0, 0)),
            pl.BlockSpec((128, 1), lambda i: (0, 0)),
        ],
        out_shape=[
            jax.ShapeDtypeStruct((K, T, 1), jnp.int32),
            jax.ShapeDtypeStruct((K, T, 1), jnp.int32),
            jax.ShapeDtypeStruct((K, T, 1), jnp.float32),
            jax.ShapeDtypeStruct((1, 1, 128), jnp.int32),
            jax.ShapeDtypeStruct((1, 1, 128), jnp.int32),
            jax.ShapeDtypeStruct((128, 1), jnp.int32),
            jax.ShapeDtypeStruct((128, 1), jnp.int32),
        ],
        scratch_shapes=[
            pltpu.VMEM((1, 128), jnp.float32),
            pltpu.VMEM((1, 128), jnp.float32),
        ],
    )(x2d, rwt)


def _dispatch_body(sel_h, rank_h, cnt0_h, offs_h, x_h, xs_h, dst_h,
                   selc_v, rankc_v, c0_v, offs_v, dst_v, tok_v,
                   rows_v, sem1, sem2):
    wid = lax.axis_index("s") * 2 + lax.axis_index("c")
    kflag = wid // 16
    tb = (wid % 16) * 128
    pb = wid * 128
    pltpu.sync_copy(sel_h.at[pl.ds(pb, 128)], selc_v)
    pltpu.sync_copy(rank_h.at[pl.ds(pb, 128)], rankc_v)
    pltpu.sync_copy(cnt0_h.at[pl.ds(0, 16)], c0_v)
    pltpu.sync_copy(offs_h.at[pl.ds(0, 16)], offs_v)
    kvec = jnp.full((16,), kflag, dtype=jnp.int32)
    for j in range(8):
        s16 = selc_v[pl.ds(j * 16, 16)]
        r16 = rankc_v[pl.ds(j * 16, 16)]
        o16 = plsc.load_gather(offs_v, [s16])
        c016 = plsc.load_gather(c0_v, [s16])
        d16 = o16 + c016 * kvec + r16
        dst_v[pl.ds(j * 16, 16)] = d16
        tok_v[pl.ds(j * 16, 16)] = tb + j * 16 + lax.iota(jnp.int32, 16)
    pltpu.async_copy(x_h.at[tok_v], rows_v, sem1).wait()
    pltpu.async_copy(rows_v, xs_h.at[dst_v], sem2).wait()
    pltpu.sync_copy(dst_v, dst_h.at[pl.ds(pb, 128)])


def _dispatch(sel_flat, rank_flat, cnt0_last, offs_last, x2d):
    f = pl.kernel(
        _dispatch_body,
        out_type=[
            jax.ShapeDtypeStruct((P, D), jnp.float32),
            jax.ShapeDtypeStruct((NPAIR,), jnp.int32),
        ],
        mesh=plsc.VectorSubcoreMesh(core_axis_name="c", subcore_axis_name="s"),
        compiler_params=pltpu.CompilerParams(needs_layout_passes=False),
        scratch_types=[
            pltpu.VMEM((128,), jnp.int32),
            pltpu.VMEM((128,), jnp.int32),
            pltpu.VMEM((16,), jnp.int32),
            pltpu.VMEM((16,), jnp.int32),
            pltpu.VMEM((128,), jnp.int32),
            pltpu.VMEM((128,), jnp.int32),
            pltpu.VMEM((128, D), jnp.float32),
            pltpu.SemaphoreType.DMA,
            pltpu.SemaphoreType.DMA,
        ],
    )
    return f(sel_flat, rank_flat, cnt0_last, offs_last, x2d)


def _ffn_body(be_ref, act_ref, xs_ref, w1_ref, w2_ref, y_ref):
    b = pl.program_id(0)

    @pl.when(act_ref[b] == 1)
    def _():
        h = jnp.dot(xs_ref[...], w1_ref[...], preferred_element_type=jnp.float32)
        h = jax.nn.gelu(h)
        y_ref[...] = jnp.dot(h, w2_ref[...], preferred_element_type=jnp.float32)


def _ffn(be, act, xs, w1, w2):
    grid_spec = pltpu.PrefetchScalarGridSpec(
        num_scalar_prefetch=2,
        grid=(NB,),
        in_specs=[
            pl.BlockSpec((BS, D), lambda b, be_r, act_r: (b, 0)),
            pl.BlockSpec((D, DFFN), lambda b, be_r, act_r: (0, be_r[b])),
            pl.BlockSpec((DFFN, D), lambda b, be_r, act_r: (be_r[b], 0)),
        ],
        out_specs=pl.BlockSpec((BS, D), lambda b, be_r, act_r: (b, 0)),
    )
    return pl.pallas_call(
        _ffn_body,
        grid_spec=grid_spec,
        out_shape=jax.ShapeDtypeStruct((P, D), jnp.float32),
    )(be, act, xs, w1, w2)


def _combine_body(y_h, dst_h, rw_h, o_h, i0_v, i1_v, w0_v, w1_v,
                  r0_v, r1_v, sem1, sem2):
    wid = lax.axis_index("s") * 2 + lax.axis_index("c")
    tb = wid * 64
    pltpu.sync_copy(dst_h.at[pl.ds(tb, 64)], i0_v)
    pltpu.sync_copy(dst_h.at[pl.ds(T + tb, 64)], i1_v)
    pltpu.sync_copy(rw_h.at[pl.ds(tb, 64)], w0_v)
    pltpu.sync_copy(rw_h.at[pl.ds(T + tb, 64)], w1_v)
    pltpu.async_copy(y_h.at[i0_v], r0_v, sem1).wait()
    pltpu.async_copy(y_h.at[i1_v], r1_v, sem2).wait()

    def body(j, carry):
        jv = jnp.full((16,), j, dtype=jnp.int32)
        w0 = plsc.load_gather(w0_v, [jv])
        w1s = plsc.load_gather(w1_v, [jv])
        for c in range(D // 16):
            sl = pl.ds(c * 16, 16)
            r0_v[j, sl] = r0_v[j, sl] * w0 + r1_v[j, sl] * w1s
        return carry

    lax.fori_loop(0, 64, body, 0)
    pltpu.sync_copy(r0_v, o_h.at[pl.ds(tb, 64)])


def _combine(y, dst, rw_flat):
    f = pl.kernel(
        _combine_body,
        out_type=jax.ShapeDtypeStruct((T, D), jnp.float32),
        mesh=plsc.VectorSubcoreMesh(core_axis_name="c", subcore_axis_name="s"),
        compiler_params=pltpu.CompilerParams(needs_layout_passes=False),
        scratch_types=[
            pltpu.VMEM((64,), jnp.int32),
            pltpu.VMEM((64,), jnp.int32),
            pltpu.VMEM((64,), jnp.float32),
            pltpu.VMEM((64,), jnp.float32),
            pltpu.VMEM((64, D), jnp.float32),
            pltpu.VMEM((64, D), jnp.float32),
            pltpu.SemaphoreType.DMA,
            pltpu.SemaphoreType.DMA,
        ],
    )
    return f(y, dst, rw_flat)


def kernel(x, router_w, w1, w2):
    b, s, d = x.shape
    x2d = x.reshape(T, D)
    rwt = router_w.T
    sel_all, rank_all, rw_all, cnt0, offs, becol, actcol = _router(x2d, rwt)
    sel_flat = sel_all.reshape(NPAIR)
    rank_flat = rank_all.reshape(NPAIR)
    rw_flat = rw_all.reshape(NPAIR)
    xs, dst = _dispatch(sel_flat, rank_flat, cnt0.reshape(128),
                        offs.reshape(128), x2d)
    y = _ffn(becol[:NB, 0], actcol[:NB, 0], xs, w1, w2)
    out = _combine(y, dst, rw_flat)
    return out.reshape(b, s, d)
